# Initial kernel scaffold; baseline (speedup 1.0000x reference)
import jax
import jax.numpy as jnp
from jax.experimental import pallas as pl


def kernel(x0, edge_index, edge_attr, z, canonical, Zemb, Wc, Wmsg0, Wedge0, Wself0, Wg0, Wmsg1, Wedge1, Wself1, Wg1):
    raise NotImplementedError("write your pallas kernel here")


# speedup vs baseline: 1.6320x; 1.6320x over previous
import functools

import jax
import jax.numpy as jnp
from jax import lax
from jax.experimental import pallas as pl
from jax.experimental.pallas import tpu as pltpu
from jax.experimental.pallas import tpu_sc as plsc

N = 10000
NPAD = 10240
E = 320000
D_EDGE = 16
INV_NEIGH = 1.0 / 3.0

CHUNK = 128
NCHUNKS = 157
ETILE = NCHUNKS * CHUNK
EPAD = 16 * ETILE
BE = 2048
RB = 2000

F32 = jnp.float32


HQ = 80


def _sc_message_pass(nq):
    per_core = nq // 2
    rows_slice = NPAD // 16
    mesh = plsc.VectorSubcoreMesh(core_axis_name="c", subcore_axis_name="s")

    @functools.partial(
        pl.kernel,
        mesh=mesh,
        compiler_params=pltpu.CompilerParams(use_tc_tiling_on_sc=False),
        out_type=jax.ShapeDtypeStruct((nq, NPAD, HQ), F32),
        scratch_types=[
            pltpu.VMEM((CHUNK,), jnp.int32),
            pltpu.VMEM((CHUNK,), jnp.int32),
            pltpu.VMEM((CHUNK, HQ), F32),
            pltpu.VMEM((CHUNK, HQ), F32),
            pltpu.SemaphoreType.DMA,
            pltpu.VMEM_SHARED((NPAD, HQ), F32),
        ],
    )
    def msg(srcq_hbm, dst_hbm, ycat_hbm, ecat_hbm, out_hbm,
            sidx_v, didx_v, rows_v, e_v, sem_g, acc_sh):
        cid = lax.axis_index("c")
        sid = lax.axis_index("s")
        ebase = sid * ETILE

        def zrow(r, carry):
            for j in range(HQ // 16):
                rows_v[r, pl.ds(j * 16, 16)] = jnp.zeros((16,), F32)
            return carry

        def mulrow(r, c2):
            for j in range(HQ // 16):
                sl = pl.ds(j * 16, 16)
                rows_v[r, sl] = rows_v[r, sl] * e_v[r, sl]
            return c2

        for p in range(per_core):
            q = cid * per_core + p
            lax.fori_loop(0, CHUNK, zrow, None)
            for k in range(5):
                pltpu.sync_copy(
                    rows_v,
                    acc_sh.at[pl.ds(sid * rows_slice + k * CHUNK, CHUNK)])
            plsc.subcore_barrier()

            def chunk(i, carry):
                off = ebase + i * CHUNK
                pltpu.sync_copy(srcq_hbm.at[q, pl.ds(off, CHUNK)], sidx_v)
                pltpu.sync_copy(dst_hbm.at[pl.ds(off, CHUNK)], didx_v)
                pltpu.async_copy(ycat_hbm.at[sidx_v], rows_v, sem_g).wait()
                pltpu.sync_copy(ecat_hbm.at[q, pl.ds(off, CHUNK)], e_v)
                lax.fori_loop(0, CHUNK, mulrow, None)
                pltpu.sync_copy(rows_v, acc_sh.at[didx_v], add=True)
                return carry
            lax.fori_loop(0, NCHUNKS, chunk, None)

            plsc.subcore_barrier()
            for k in range(5):
                r0 = sid * rows_slice + k * CHUNK
                pltpu.sync_copy(acc_sh.at[pl.ds(r0, CHUNK)], rows_v)
                pltpu.sync_copy(rows_v, out_hbm.at[q, pl.ds(r0, CHUNK)])

    return msg


def _tc_init(x0, z2, canonical, Zemb, Wc, Wm0):
    def body(x0_r, z_r, can_r, zemb_r, wc_r, wm_r, x_r, yc_r):
        zb = z_r[...]
        onehot = (zb == lax.broadcasted_iota(jnp.int32, (RB, 100), 1)
                  ).astype(F32)
        xv = (x0_r[...]
              + jnp.dot(onehot, zemb_r[...], preferred_element_type=F32)
              + jnp.dot(can_r[...], wc_r[...], preferred_element_type=F32))
        x_r[...] = xv
        y = jnp.dot(xv, wm_r[...], preferred_element_type=F32)
        yc_r[0] = y[:, :80]
        yc_r[1] = jnp.concatenate([y[:, 80:], jnp.zeros((RB, 8), F32)], axis=1)

    return pl.pallas_call(
        body,
        grid=(N // RB,),
        in_specs=[
            pl.BlockSpec((RB, 152), lambda i: (i, 0)),
            pl.BlockSpec((RB, 1), lambda i: (i, 0)),
            pl.BlockSpec((RB, 3), lambda i: (i, 0)),
            pl.BlockSpec((100, 152), lambda i: (0, 0)),
            pl.BlockSpec((3, 152), lambda i: (0, 0)),
            pl.BlockSpec((152, 152), lambda i: (0, 0)),
        ],
        out_specs=[
            pl.BlockSpec((RB, 152), lambda i: (i, 0)),
            pl.BlockSpec((2, RB, 80), lambda i: (0, i, 0)),
        ],
        out_shape=[
            jax.ShapeDtypeStruct((N, 152), F32),
            jax.ShapeDtypeStruct((2, N, 80), F32),
        ],
    )(x0, z2, canonical, Zemb, Wc, Wm0)


def _tc_edge_coeffs(ea_p, We0, We1):
    def body(ea_r, w0_r, w1_r, e0_r, e1_r):
        a = ea_r[...]
        e0 = jnp.dot(a, w0_r[...], preferred_element_type=F32) * INV_NEIGH
        e0_r[0] = e0[:, :80]
        e0_r[1] = jnp.concatenate([e0[:, 80:], jnp.zeros((BE, 8), F32)],
                                  axis=1)
        e1 = jnp.dot(a, w1_r[...], preferred_element_type=F32) * INV_NEIGH
        e1_r[0] = e1[:, :80]
        e1_r[1] = e1[:, 80:160]
        e1_r[2] = e1[:, 160:240]
        e1_r[3] = jnp.concatenate([e1[:, 240:], jnp.zeros((BE, 16), F32)],
                                  axis=1)

    return pl.pallas_call(
        body,
        grid=(EPAD // BE,),
        in_specs=[
            pl.BlockSpec((BE, D_EDGE), lambda i: (i, 0)),
            pl.BlockSpec((D_EDGE, 152), lambda i: (0, 0)),
            pl.BlockSpec((D_EDGE, 304), lambda i: (0, 0)),
        ],
        out_specs=[
            pl.BlockSpec((2, BE, 80), lambda i: (0, i, 0)),
            pl.BlockSpec((4, BE, 80), lambda i: (0, i, 0)),
        ],
        out_shape=[
            jax.ShapeDtypeStruct((2, EPAD, 80), F32),
            jax.ShapeDtypeStruct((4, EPAD, 80), F32),
        ],
    )(ea_p, We0, We1)


def _tc_mid(agg0c, x, Wg0, Ws0, Wm1):
    def body(ag_r, x_r, wg_r, ws_r, wm_r, x1_r, yc_r):
        agg = jnp.concatenate([ag_r[0], ag_r[1][:, :72]], axis=1)
        gate = jax.nn.sigmoid(jnp.dot(agg, wg_r[...],
                                      preferred_element_type=F32))
        x1 = jnp.dot(x_r[...], ws_r[...],
                     preferred_element_type=F32) + agg * gate
        x1_r[...] = x1
        y1 = jnp.dot(x1, wm_r[...], preferred_element_type=F32)
        yc_r[0] = y1[:, :80]
        yc_r[1] = y1[:, 80:160]
        yc_r[2] = y1[:, 160:240]
        yc_r[3] = jnp.concatenate([y1[:, 240:], jnp.zeros((RB, 16), F32)],
                                  axis=1)

    return pl.pallas_call(
        body,
        grid=(N // RB,),
        in_specs=[
            pl.BlockSpec((2, RB, 80), lambda i: (0, i, 0)),
            pl.BlockSpec((RB, 152), lambda i: (i, 0)),
            pl.BlockSpec((152, 152), lambda i: (0, 0)),
            pl.BlockSpec((152, 152), lambda i: (0, 0)),
            pl.BlockSpec((152, 304), lambda i: (0, 0)),
        ],
        out_specs=[
            pl.BlockSpec((RB, 152), lambda i: (i, 0)),
            pl.BlockSpec((4, RB, 80), lambda i: (0, i, 0)),
        ],
        out_shape=[
            jax.ShapeDtypeStruct((N, 152), F32),
            jax.ShapeDtypeStruct((4, N, 80), F32),
        ],
    )(agg0c, x, Wg0, Ws0, Wm1)


def _tc_final(agg1c, x1, Wg1, Ws1):
    def body(ag_r, x1_r, wg_r, ws_r, out_r):
        agg = jnp.concatenate(
            [ag_r[0], ag_r[1], ag_r[2], ag_r[3][:, :64]], axis=1)
        gate = jax.nn.sigmoid(jnp.dot(agg, wg_r[...],
                                      preferred_element_type=F32))
        out_r[...] = jnp.dot(x1_r[...], ws_r[...],
                             preferred_element_type=F32) + agg * gate

    return pl.pallas_call(
        body,
        grid=(N // RB,),
        in_specs=[
            pl.BlockSpec((4, RB, 80), lambda i: (0, i, 0)),
            pl.BlockSpec((RB, 152), lambda i: (i, 0)),
            pl.BlockSpec((304, 304), lambda i: (0, 0)),
            pl.BlockSpec((152, 304), lambda i: (0, 0)),
        ],
        out_specs=pl.BlockSpec((RB, 304), lambda i: (i, 0)),
        out_shape=jax.ShapeDtypeStruct((N, 304), F32),
    )(agg1c, x1, Wg1, Ws1)


@jax.jit
def kernel(x0, edge_index, edge_attr, z, canonical, Zemb, Wc,
           Wmsg0, Wedge0, Wself0, Wg0, Wmsg1, Wedge1, Wself1, Wg1):
    src = edge_index[0].astype(jnp.int32)
    dst = edge_index[1].astype(jnp.int32)
    pad = EPAD - E
    src_p = jnp.pad(src, (0, pad))
    dst_p = jnp.pad(dst, (0, pad))
    ea_p = jnp.pad(edge_attr, ((0, pad), (0, 0)))
    src2 = jnp.stack([src_p, src_p + N])
    src4 = jnp.stack([src_p + q * N for q in range(4)])
    z2 = z.astype(jnp.int32).reshape(N, 1)

    x, y0c = _tc_init(x0, z2, canonical, Zemb, Wc, Wmsg0)
    e0c, e1c = _tc_edge_coeffs(ea_p, Wedge0, Wedge1)

    agg0c = _sc_message_pass(2)(src2, dst_p, y0c.reshape(2 * N, HQ), e0c)
    x1, y1c = _tc_mid(agg0c, x, Wg0, Wself0, Wmsg1)
    agg1c = _sc_message_pass(4)(src4, dst_p, y1c.reshape(4 * N, HQ), e1c)
    return _tc_final(agg1c, x1, Wg1, Wself1)


# speedup vs baseline: 1.8130x; 1.1109x over previous
import functools

import jax
import jax.numpy as jnp
from jax import lax
from jax.experimental import pallas as pl
from jax.experimental.pallas import tpu as pltpu
from jax.experimental.pallas import tpu_sc as plsc

N = 10000
NPAD = 10240
E = 320000
D_EDGE = 16
INV_NEIGH = 1.0 / 3.0

CHUNK = 128
NCHUNKS = 160
ETILE = NCHUNKS * CHUNK
EPAD = 16 * ETILE
BE = 2048
RB = 2000

F32 = jnp.float32


HQ = 80


def _sc_message_pass(nq):
    per_core = nq // 2
    rows_slice = NPAD // 16
    mesh = plsc.VectorSubcoreMesh(core_axis_name="c", subcore_axis_name="s")

    @functools.partial(
        pl.kernel,
        mesh=mesh,
        compiler_params=pltpu.CompilerParams(use_tc_tiling_on_sc=False),
        out_type=jax.ShapeDtypeStruct((nq, NPAD, HQ), F32),
        scratch_types=[
            [pltpu.VMEM((CHUNK,), jnp.int32)] * 2,
            [pltpu.VMEM((CHUNK,), jnp.int32)] * 2,
            pltpu.VMEM((2, CHUNK, HQ), F32),
            pltpu.VMEM((2, CHUNK, HQ), F32),
            [pltpu.SemaphoreType.DMA] * 6,
            pltpu.VMEM_SHARED((NPAD, HQ), F32),
        ],
    )
    def msg(srcq_hbm, dst_hbm, ycat_hbm, ecat_hbm, out_hbm,
            sidx_b, didx_b, rows_v, e_v, sems, acc_sh):
        sem_g = sems[:2]
        sem_e = sems[2:4]
        sem_i = sems[4:]
        cid = lax.axis_index("c")
        sid = lax.axis_index("s")
        ebase = sid * ETILE

        def zrow(r, carry):
            for j in range(HQ // 16):
                rows_v[0, r, pl.ds(j * 16, 16)] = jnp.zeros((16,), F32)
            return carry

        def mulrow(b):
            def go(r, c2):
                for j in range(HQ // 16):
                    sl = pl.ds(j * 16, 16)
                    rows_v[b, r, sl] = rows_v[b, r, sl] * e_v[b, r, sl]
                return c2
            return go

        def issue_idx(i, b, q):
            pltpu.async_copy(srcq_hbm.at[q, sid, i], sidx_b[b], sem_i[b])
            pltpu.async_copy(dst_hbm.at[sid, i], didx_b[b], sem_i[b])

        def wait_idx(b, q):
            pltpu.make_async_copy(srcq_hbm.at[q, sid, 0], sidx_b[b],
                                  sem_i[b]).wait()
            pltpu.make_async_copy(dst_hbm.at[sid, 0], didx_b[b],
                                  sem_i[b]).wait()

        def fetch(i, b, q):
            pltpu.async_copy(ycat_hbm.at[sidx_b[b]], rows_v.at[b], sem_g[b])
            pltpu.async_copy(
                ecat_hbm.at[q, pl.ds(ebase + i * CHUNK, CHUNK)],
                e_v.at[b], sem_e[b])

        def wait_fetch(b):
            pltpu.make_async_copy(ycat_hbm.at[sidx_b[b]], rows_v.at[b],
                                  sem_g[b]).wait()
            pltpu.make_async_copy(
                ecat_hbm.at[0, pl.ds(0, CHUNK)], e_v.at[b], sem_e[b]).wait()

        def scatter(b):
            pltpu.sync_copy(rows_v.at[b], acc_sh.at[didx_b[b]], add=True)

        for p in range(per_core):
            q = cid * per_core + p
            lax.fori_loop(0, CHUNK, zrow, None)
            for k in range(5):
                pltpu.sync_copy(
                    rows_v.at[0],
                    acc_sh.at[pl.ds(sid * rows_slice + k * CHUNK, CHUNK)])
            plsc.subcore_barrier()

            pltpu.sync_copy(srcq_hbm.at[q, sid, 0], sidx_b[0])
            pltpu.sync_copy(dst_hbm.at[sid, 0], didx_b[0])
            fetch(0, 0, q)
            issue_idx(1, 1, q)

            def group(g, carry):
                for b in range(2):
                    l = 2 * g + b
                    nb = 1 - b
                    @pl.when(l + 1 < NCHUNKS)
                    def _():
                        wait_idx(nb, q)
                        fetch(l + 1, nb, q)
                    wait_fetch(b)
                    lax.fori_loop(0, CHUNK, mulrow(b), None)
                    scatter(b)
                    @pl.when(l + 2 < NCHUNKS)
                    def _():
                        issue_idx(l + 2, b, q)
                return carry
            lax.fori_loop(0, NCHUNKS // 2, group, None)

            plsc.subcore_barrier()
            for k in range(5):
                r0 = sid * rows_slice + k * CHUNK
                pltpu.sync_copy(acc_sh.at[pl.ds(r0, CHUNK)], rows_v.at[0])
                pltpu.sync_copy(rows_v.at[0], out_hbm.at[q, pl.ds(r0, CHUNK)])

    return msg


def _tc_init(x0, z2, canonical, Zemb, Wc, Wm0):
    def body(x0_r, z_r, can_r, zemb_r, wc_r, wm_r, x_r, yc_r):
        zb = z_r[...]
        onehot = (zb == lax.broadcasted_iota(jnp.int32, (RB, 100), 1)
                  ).astype(F32)
        xv = (x0_r[...]
              + jnp.dot(onehot, zemb_r[...], preferred_element_type=F32)
              + jnp.dot(can_r[...], wc_r[...], preferred_element_type=F32))
        x_r[...] = xv
        y = jnp.dot(xv, wm_r[...], preferred_element_type=F32)
        yc_r[0] = y[:, :80]
        yc_r[1] = jnp.concatenate([y[:, 80:], jnp.zeros((RB, 8), F32)], axis=1)

    return pl.pallas_call(
        body,
        grid=(N // RB,),
        in_specs=[
            pl.BlockSpec((RB, 152), lambda i: (i, 0)),
            pl.BlockSpec((RB, 1), lambda i: (i, 0)),
            pl.BlockSpec((RB, 3), lambda i: (i, 0)),
            pl.BlockSpec((100, 152), lambda i: (0, 0)),
            pl.BlockSpec((3, 152), lambda i: (0, 0)),
            pl.BlockSpec((152, 152), lambda i: (0, 0)),
        ],
        out_specs=[
            pl.BlockSpec((RB, 152), lambda i: (i, 0)),
            pl.BlockSpec((2, RB, 80), lambda i: (0, i, 0)),
        ],
        out_shape=[
            jax.ShapeDtypeStruct((N, 152), F32),
            jax.ShapeDtypeStruct((2, N, 80), F32),
        ],
    )(x0, z2, canonical, Zemb, Wc, Wm0)


def _tc_edge_coeffs(ea_p, We0, We1):
    def body(ea_r, w0_r, w1_r, e0_r, e1_r):
        a = ea_r[...]
        e0 = jnp.dot(a, w0_r[...], preferred_element_type=F32) * INV_NEIGH
        e0_r[0] = e0[:, :80]
        e0_r[1] = jnp.concatenate([e0[:, 80:], jnp.zeros((BE, 8), F32)],
                                  axis=1)
        e1 = jnp.dot(a, w1_r[...], preferred_element_type=F32) * INV_NEIGH
        e1_r[0] = e1[:, :80]
        e1_r[1] = e1[:, 80:160]
        e1_r[2] = e1[:, 160:240]
        e1_r[3] = jnp.concatenate([e1[:, 240:], jnp.zeros((BE, 16), F32)],
                                  axis=1)

    return pl.pallas_call(
        body,
        grid=(EPAD // BE,),
        in_specs=[
            pl.BlockSpec((BE, D_EDGE), lambda i: (i, 0)),
            pl.BlockSpec((D_EDGE, 152), lambda i: (0, 0)),
            pl.BlockSpec((D_EDGE, 304), lambda i: (0, 0)),
        ],
        out_specs=[
            pl.BlockSpec((2, BE, 80), lambda i: (0, i, 0)),
            pl.BlockSpec((4, BE, 80), lambda i: (0, i, 0)),
        ],
        out_shape=[
            jax.ShapeDtypeStruct((2, EPAD, 80), F32),
            jax.ShapeDtypeStruct((4, EPAD, 80), F32),
        ],
    )(ea_p, We0, We1)


def _tc_mid(agg0c, x, Wg0, Ws0, Wm1):
    def body(ag_r, x_r, wg_r, ws_r, wm_r, x1_r, yc_r):
        agg = jnp.concatenate([ag_r[0], ag_r[1][:, :72]], axis=1)
        gate = jax.nn.sigmoid(jnp.dot(agg, wg_r[...],
                                      preferred_element_type=F32))
        x1 = jnp.dot(x_r[...], ws_r[...],
                     preferred_element_type=F32) + agg * gate
        x1_r[...] = x1
        y1 = jnp.dot(x1, wm_r[...], preferred_element_type=F32)
        yc_r[0] = y1[:, :80]
        yc_r[1] = y1[:, 80:160]
        yc_r[2] = y1[:, 160:240]
        yc_r[3] = jnp.concatenate([y1[:, 240:], jnp.zeros((RB, 16), F32)],
                                  axis=1)

    return pl.pallas_call(
        body,
        grid=(N // RB,),
        in_specs=[
            pl.BlockSpec((2, RB, 80), lambda i: (0, i, 0)),
            pl.BlockSpec((RB, 152), lambda i: (i, 0)),
            pl.BlockSpec((152, 152), lambda i: (0, 0)),
            pl.BlockSpec((152, 152), lambda i: (0, 0)),
            pl.BlockSpec((152, 304), lambda i: (0, 0)),
        ],
        out_specs=[
            pl.BlockSpec((RB, 152), lambda i: (i, 0)),
            pl.BlockSpec((4, RB, 80), lambda i: (0, i, 0)),
        ],
        out_shape=[
            jax.ShapeDtypeStruct((N, 152), F32),
            jax.ShapeDtypeStruct((4, N, 80), F32),
        ],
    )(agg0c, x, Wg0, Ws0, Wm1)


def _tc_final(agg1c, x1, Wg1, Ws1):
    def body(ag_r, x1_r, wg_r, ws_r, out_r):
        agg = jnp.concatenate(
            [ag_r[0], ag_r[1], ag_r[2], ag_r[3][:, :64]], axis=1)
        gate = jax.nn.sigmoid(jnp.dot(agg, wg_r[...],
                                      preferred_element_type=F32))
        out_r[...] = jnp.dot(x1_r[...], ws_r[...],
                             preferred_element_type=F32) + agg * gate

    return pl.pallas_call(
        body,
        grid=(N // RB,),
        in_specs=[
            pl.BlockSpec((4, RB, 80), lambda i: (0, i, 0)),
            pl.BlockSpec((RB, 152), lambda i: (i, 0)),
            pl.BlockSpec((304, 304), lambda i: (0, 0)),
            pl.BlockSpec((152, 304), lambda i: (0, 0)),
        ],
        out_specs=pl.BlockSpec((RB, 304), lambda i: (i, 0)),
        out_shape=jax.ShapeDtypeStruct((N, 304), F32),
    )(agg1c, x1, Wg1, Ws1)


@jax.jit
def kernel(x0, edge_index, edge_attr, z, canonical, Zemb, Wc,
           Wmsg0, Wedge0, Wself0, Wg0, Wmsg1, Wedge1, Wself1, Wg1):
    src = edge_index[0].astype(jnp.int32)
    dst = edge_index[1].astype(jnp.int32)
    pad = EPAD - E
    src_p = jnp.pad(src, (0, pad))
    dst_p = jnp.pad(dst, (0, pad))
    ea_p = jnp.pad(edge_attr, ((0, pad), (0, 0)))
    tile4 = (16, NCHUNKS, CHUNK)
    src2 = jnp.stack([src_p, src_p + N]).reshape((2,) + tile4)
    src4 = jnp.stack([src_p + q * N for q in range(4)]).reshape((4,) + tile4)
    dst_t = dst_p.reshape(tile4)
    z2 = z.astype(jnp.int32).reshape(N, 1)

    x, y0c = _tc_init(x0, z2, canonical, Zemb, Wc, Wmsg0)
    e0c, e1c = _tc_edge_coeffs(ea_p, Wedge0, Wedge1)

    agg0c = _sc_message_pass(2)(src2, dst_t, y0c.reshape(2 * N, HQ), e0c)
    x1, y1c = _tc_mid(agg0c, x, Wg0, Wself0, Wmsg1)
    agg1c = _sc_message_pass(4)(src4, dst_t, y1c.reshape(4 * N, HQ), e1c)
    return _tc_final(agg1c, x1, Wg1, Wself1)
